# baseline (speedup 1.0000x reference)
import jax
import jax.numpy as jnp
from jax import lax
from jax.experimental import pallas as pl
from jax.experimental.pallas import tpu as pltpu
from jax.experimental.pallas import tpu_sc as plsc

B = 4096
L = 200
D = 32

NC = 2
NS = 16
NW = NC * NS

B_PER_W = B // NW
R = 4
N_CHUNKS = B_PER_W // R
SPLITS = ((0, 128), (128, 72))


def _body(tp_t, ent_t, val_t, ha_t, tp_i, ent_i, val_i, ha_i,
          tp_o, ent_o, val_o, ha_o,
          idx0, idx1, rows0, rows1, gsem0, gsem1, ssem0, ssem1):
    wid = lax.axis_index("s") * NC + lax.axis_index("c")
    b_base = wid * B_PER_W
    idx_v = (idx0, idx1)
    rows_v = (rows0, rows1)
    gsem = (gsem0, gsem1)
    ssem = (ssem0, ssem1)

    for tab, idx, out in ((tp_t, tp_i, tp_o), (ent_t, ent_i, ent_o),
                          (val_t, val_i, val_o), (ha_t, ha_i, ha_o)):

        def issue_gathers(cc, b, tab=tab, idx=idx):
            b0 = b_base + cc * R
            pltpu.sync_copy(idx.at[pl.ds(b0, R)], idx_v[b])
            for i in range(R):
                for off, n in SPLITS:
                    pltpu.async_copy(
                        tab.at[idx_v[b].at[i, pl.ds(off, n)]],
                        rows_v[b].at[i, pl.ds(off, n)],
                        gsem[b],
                    )

        def wait_gathers(b, out=out):
            pltpu.make_async_copy(
                out.at[pl.ds(0, R)], rows_v[b], gsem[b]
            ).wait()

        def issue_store(cc, b, out=out):
            b0 = b_base + cc * R
            pltpu.async_copy(rows_v[b], out.at[pl.ds(b0, R)], ssem[b])

        def wait_store(b, out=out):
            pltpu.make_async_copy(
                rows_v[b], out.at[pl.ds(0, R)], ssem[b]
            ).wait()

        issue_gathers(0, 0)
        issue_gathers(1, 1)

        @pl.loop(0, N_CHUNKS - 2, step=2)
        def _steady(c):
            for b in range(2):
                cc = c + b
                wait_gathers(b)
                issue_store(cc, b)
                wait_store(b)
                issue_gathers(cc + 2, b)

        for b in range(2):
            wait_gathers(b)
            issue_store(N_CHUNKS - 2 + b, b)
        for b in range(2):
            wait_store(b)


def kernel(tp, ent, val, ha, tp_table, ent_table, val_table, ha_table):
    mesh = plsc.VectorSubcoreMesh(core_axis_name="c", subcore_axis_name="s")
    out_sd = jax.ShapeDtypeStruct((B, L, D), jnp.float32)
    fn = pl.kernel(
        _body,
        out_type=(out_sd, out_sd, out_sd, out_sd),
        mesh=mesh,
        scratch_types=[
            pltpu.VMEM((R, L), jnp.int32),
            pltpu.VMEM((R, L), jnp.int32),
            pltpu.VMEM((R, L, D), jnp.float32),
            pltpu.VMEM((R, L, D), jnp.float32),
            pltpu.SemaphoreType.DMA,
            pltpu.SemaphoreType.DMA,
            pltpu.SemaphoreType.DMA,
            pltpu.SemaphoreType.DMA,
        ],
        compiler_params=pltpu.CompilerParams(use_tc_tiling_on_sc=False),
    )
    return fn(tp_table, ent_table, val_table, ha_table, tp, ent, val, ha)
